# Initial kernel scaffold; baseline (speedup 1.0000x reference)
import jax
import jax.numpy as jnp
from jax.experimental import pallas as pl


def kernel(x, edge_index, W1, b1, W2, b2):
    raise NotImplementedError("write your pallas kernel here")


# speedup vs baseline: 16.4908x; 16.4908x over previous
import functools

import jax
import jax.numpy as jnp
from jax import lax
from jax.experimental import pallas as pl
from jax.experimental.pallas import tpu as pltpu
from jax.experimental.pallas import tpu_sc as plsc

NC = 2
NS = 16
NW = NC * NS
CH = 128
BM = 1024


def _sc_mesh():
    return plsc.VectorSubcoreMesh(core_axis_name="c", subcore_axis_name="s")


def _make_deg_kernel(NP, nchunk):
    rpt = NP // NS

    @functools.partial(
        pl.kernel,
        out_type=jax.ShapeDtypeStruct((NC, NP), jnp.float32),
        mesh=_sc_mesh(),
        scratch_types=[
            pltpu.VMEM((nchunk, CH), jnp.int32),
            pltpu.VMEM((CH,), jnp.float32),
            pltpu.VMEM_SHARED((NP,), jnp.float32),
        ],
    )
    def deg_kernel(dst_hbm, ones_hbm, zer1_hbm, out_hbm, dstv, ones_v, acc):
        c = lax.axis_index("c")
        s = lax.axis_index("s")
        w = c * NS + s
        pltpu.sync_copy(dst_hbm.at[w], dstv)
        pltpu.sync_copy(ones_hbm, ones_v)
        pltpu.sync_copy(zer1_hbm, acc.at[pl.ds(s * rpt, rpt)])
        plsc.subcore_barrier()

        def body(j, carry):
            pltpu.sync_copy(ones_v, acc.at[dstv.at[j]], add=True)
            return carry

        lax.fori_loop(0, nchunk, body, 0)
        plsc.subcore_barrier()
        pltpu.sync_copy(acc.at[pl.ds(s * rpt, rpt)],
                        out_hbm.at[c].at[pl.ds(s * rpt, rpt)])

    return deg_kernel


def _make_agg_kernel(NP, D, nchunk):
    rpt = NP // NS

    @functools.partial(
        pl.kernel,
        out_type=jax.ShapeDtypeStruct((NC, NP, D), jnp.float32),
        mesh=_sc_mesh(),
        scratch_types=[
            pltpu.VMEM((nchunk, CH), jnp.int32),
            pltpu.VMEM((nchunk, CH), jnp.int32),
            pltpu.VMEM((CH, D), jnp.float32),
            pltpu.VMEM_SHARED((NP, D), jnp.float32),
            pltpu.SemaphoreType.DMA,
        ],
    )
    def agg_kernel(xp_hbm, src_hbm, dst_hbm, zer2_hbm, out_hbm,
                   srcv, dstv, rows0, acc, sem0):
        c = lax.axis_index("c")
        s = lax.axis_index("s")
        w = c * NS + s
        pltpu.sync_copy(src_hbm.at[w], srcv)
        pltpu.sync_copy(dst_hbm.at[w], dstv)
        pltpu.sync_copy(zer2_hbm, acc.at[pl.ds(s * rpt, rpt)])
        plsc.subcore_barrier()

        def body(j, carry):
            pltpu.async_copy(xp_hbm.at[srcv.at[j]], rows0, sem0).wait()
            pltpu.sync_copy(rows0, acc.at[dstv.at[j]], add=True)
            return carry

        lax.fori_loop(0, nchunk, body, 0)
        plsc.subcore_barrier()
        pltpu.sync_copy(acc.at[pl.ds(s * rpt, rpt)],
                        out_hbm.at[c].at[pl.ds(s * rpt, rpt)])

    return agg_kernel


def _make_sagg_kernel(NP, nchunk):
    rpt = NP // NS

    @functools.partial(
        pl.kernel,
        out_type=jax.ShapeDtypeStruct((NC, NP), jnp.float32),
        mesh=_sc_mesh(),
        scratch_types=[
            pltpu.VMEM((nchunk, CH), jnp.int32),
            pltpu.VMEM((nchunk, CH), jnp.int32),
            pltpu.VMEM((CH,), jnp.float32),
            pltpu.VMEM_SHARED((NP,), jnp.float32),
            pltpu.SemaphoreType.DMA,
        ],
    )
    def sagg_kernel(gp_hbm, src_hbm, dst_hbm, zer1_hbm, out_hbm,
                    srcv, dstv, vals, acc, sem0):
        c = lax.axis_index("c")
        s = lax.axis_index("s")
        w = c * NS + s
        pltpu.sync_copy(src_hbm.at[w], srcv)
        pltpu.sync_copy(dst_hbm.at[w], dstv)
        pltpu.sync_copy(zer1_hbm, acc.at[pl.ds(s * rpt, rpt)])
        plsc.subcore_barrier()

        def body(j, carry):
            pltpu.async_copy(gp_hbm.at[srcv.at[j]], vals, sem0).wait()
            pltpu.sync_copy(vals, acc.at[dstv.at[j]], add=True)
            return carry

        lax.fori_loop(0, nchunk, body, 0)
        plsc.subcore_barrier()
        pltpu.sync_copy(acc.at[pl.ds(s * rpt, rpt)],
                        out_hbm.at[c].at[pl.ds(s * rpt, rpt)])

    return sagg_kernel


def _tc1_body(d0, d1, xin, dis_o, xp_o):
    deg = d0[...] + d1[...] + 1.0
    dis = lax.rsqrt(deg)
    dis_o[...] = dis
    xp_o[...] = xin[...] * dis[:, None]


def _tc2_body(p0, p1, xp, dis, w1, b1r, w2r, gp_o):
    dis_v = dis[...]
    t = (p0[...] + p1[...] + xp[...]) * dis_v[:, None]
    h = jnp.dot(t, w1[...], preferred_element_type=jnp.float32) + b1r[...][None, :]
    r = jnp.maximum(h, 0.0)
    g = jnp.sum(r * w2r[...][None, :], axis=1)
    gp_o[...] = g * dis_v


def _tc3_body(q0, q1, gp, dis, b2v, o):
    o[...] = dis[...] * (q0[...] + q1[...] + gp[...]) + b2v[...]


def kernel(x, edge_index, W1, b1, W2, b2):
    N, D = x.shape
    E = edge_index.shape[1]
    NP = pl.cdiv(N + 1, BM) * BM
    nchunk = pl.cdiv(E, NW * CH)
    nchunk = nchunk + (nchunk % 2)
    EP = NW * nchunk * CH
    rpt = NP // NS

    src = edge_index[0].astype(jnp.int32)
    dst = edge_index[1].astype(jnp.int32)
    padi = jnp.full((EP - E,), NP - 1, jnp.int32)
    srcr = jnp.concatenate([src, padi]).reshape(NW, nchunk, CH)
    dstr = jnp.concatenate([dst, padi]).reshape(NW, nchunk, CH)

    ones = jnp.ones((CH,), jnp.float32)
    zer1 = jnp.zeros((rpt,), jnp.float32)
    zer2 = jnp.zeros((rpt, D), jnp.float32)
    xin = jnp.pad(x, ((0, NP - N), (0, 0)))

    dp = _make_deg_kernel(NP, nchunk)(dstr, ones, zer1)

    grid = NP // BM
    dis, xp = pl.pallas_call(
        _tc1_body,
        grid=(grid,),
        in_specs=[
            pl.BlockSpec((BM,), lambda i: (i,)),
            pl.BlockSpec((BM,), lambda i: (i,)),
            pl.BlockSpec((BM, D), lambda i: (i, 0)),
        ],
        out_specs=[
            pl.BlockSpec((BM,), lambda i: (i,)),
            pl.BlockSpec((BM, D), lambda i: (i, 0)),
        ],
        out_shape=[
            jax.ShapeDtypeStruct((NP,), jnp.float32),
            jax.ShapeDtypeStruct((NP, D), jnp.float32),
        ],
    )(dp[0], dp[1], xin)

    ap = _make_agg_kernel(NP, D, nchunk)(xp, srcr, dstr, zer2)

    gp = pl.pallas_call(
        _tc2_body,
        grid=(grid,),
        in_specs=[
            pl.BlockSpec((BM, D), lambda i: (i, 0)),
            pl.BlockSpec((BM, D), lambda i: (i, 0)),
            pl.BlockSpec((BM, D), lambda i: (i, 0)),
            pl.BlockSpec((BM,), lambda i: (i,)),
            pl.BlockSpec((D, D), lambda i: (0, 0)),
            pl.BlockSpec((D,), lambda i: (0,)),
            pl.BlockSpec((D,), lambda i: (0,)),
        ],
        out_specs=pl.BlockSpec((BM,), lambda i: (i,)),
        out_shape=jax.ShapeDtypeStruct((NP,), jnp.float32),
    )(ap[0], ap[1], xp, dis, W1, b1, W2[:, 0])

    qp = _make_sagg_kernel(NP, nchunk)(gp, srcr, dstr, zer1)

    b2v = jnp.broadcast_to(b2, (NP,))
    outf = pl.pallas_call(
        _tc3_body,
        grid=(1,),
        in_specs=[pl.BlockSpec((NP,), lambda i: (0,))] * 5,
        out_specs=pl.BlockSpec((NP,), lambda i: (0,)),
        out_shape=jax.ShapeDtypeStruct((NP,), jnp.float32),
    )(qp[0], qp[1], gp, dis, b2v)

    return outf[:N, None]
